# Initial kernel scaffold; baseline (speedup 1.0000x reference)
import jax
import jax.numpy as jnp
from jax.experimental import pallas as pl


def kernel(input, table):
    raise NotImplementedError("write your pallas kernel here")


# speedup vs baseline: 1.1868x; 1.1868x over previous
import functools

import jax
import jax.numpy as jnp
from jax import lax
from jax.experimental import pallas as pl
from jax.experimental.pallas import tpu as pltpu
from jax.experimental.pallas import tpu_sc as plsc

VOCAB_DIM = 1000000
D = 32
NW = 32
CHUNK = 128


def _build(B):
    b_per_w = B // NW
    nchunk = b_per_w // CHUNK
    mesh = plsc.VectorSubcoreMesh(core_axis_name="c", subcore_axis_name="s")

    @functools.partial(
        pl.kernel,
        mesh=mesh,
        out_type=jax.ShapeDtypeStruct((NW, nchunk, CHUNK, D), jnp.float32),
        scratch_types=[
            pltpu.VMEM((nchunk, CHUNK), jnp.int32),
            pltpu.VMEM((CHUNK, D), jnp.float32),
            pltpu.SemaphoreType.DMA,
        ],
        compiler_params=pltpu.CompilerParams(use_tc_tiling_on_sc=False),
    )
    def k(idx_hbm, table_hbm, out_hbm, idx_v, rows_v, sem):
        wid = lax.axis_index("s") * 2 + lax.axis_index("c")
        pltpu.sync_copy(idx_hbm.at[wid], idx_v)

        def body(c, carry):
            pltpu.async_copy(table_hbm.at[idx_v.at[c]], rows_v, sem).wait()
            pltpu.sync_copy(rows_v, out_hbm.at[wid, c])
            return carry

        lax.fori_loop(0, nchunk, body, 0)

    return k


def kernel(input, table):
    orig_shape = input.shape
    B = input.size
    idx = input.reshape(NW, B // NW // CHUNK, CHUNK).astype(jnp.int32)
    out = _build(B)(idx, table)
    return out.reshape(orig_shape + (D,))


# speedup vs baseline: 1.3081x; 1.1022x over previous
import functools

import jax
import jax.numpy as jnp
from jax import lax
from jax.experimental import pallas as pl
from jax.experimental.pallas import tpu as pltpu
from jax.experimental.pallas import tpu_sc as plsc

D = 32
NW = 32
CHUNK = 128
NBUF = 8
K = 4


def _build(B):
    b_per_w = B // NW
    nchunk = b_per_w // CHUNK
    assert nchunk % NBUF == 0 and nchunk >= 2 * NBUF
    ngroups = (nchunk - NBUF) // NBUF
    mesh = plsc.VectorSubcoreMesh(core_axis_name="c", subcore_axis_name="s")

    @functools.partial(
        pl.kernel,
        mesh=mesh,
        out_type=jax.ShapeDtypeStruct((NW, nchunk, CHUNK, D), jnp.float32),
        scratch_types=[
            pltpu.VMEM((nchunk, CHUNK), jnp.int32),
            pltpu.VMEM((NBUF, CHUNK, D), jnp.float32),
        ] + [pltpu.SemaphoreType.DMA] * (2 * NBUF),
        compiler_params=pltpu.CompilerParams(use_tc_tiling_on_sc=False),
    )
    def k(idx_hbm, table_hbm, out_hbm, idx_v, rows_v, *sems):
        gsem = sems[:NBUF]
        osem = sems[NBUF:]
        wid = lax.axis_index("s") * 2 + lax.axis_index("c")
        pltpu.sync_copy(idx_hbm.at[wid], idx_v)

        def gather_start(c, b):
            pltpu.make_async_copy(
                table_hbm.at[idx_v.at[c]], rows_v.at[b], gsem[b]).start()

        def gather_wait(b):
            pltpu.make_async_copy(
                table_hbm.at[idx_v.at[0]], rows_v.at[b], gsem[b]).wait()

        def out_start(c, b):
            pltpu.make_async_copy(
                rows_v.at[b], out_hbm.at[wid, c], osem[b]).start()

        def out_wait(b):
            pltpu.make_async_copy(
                rows_v.at[b], out_hbm.at[wid, 0], osem[b]).wait()

        for c in range(K):
            gather_start(c, c % NBUF)

        for c in range(NBUF - K):
            gather_wait(c % NBUF)
            out_start(c, c % NBUF)
            gather_start(c + K, (c + K) % NBUF)

        @pl.loop(0, ngroups)
        def _main(g):
            c0 = (NBUF - K) + g * NBUF
            for b8 in range(NBUF):
                c = c0 + b8
                b = (NBUF - K + b8) % NBUF
                gather_wait(b)
                out_start(c, b)
                bp = b8
                out_wait(bp)
                gather_start(c + K, bp)

        for i in range(K):
            c = nchunk - K + i
            gather_wait(c % NBUF)
            out_start(c, c % NBUF)
        for b in range(NBUF):
            out_wait(b)

    return k


def kernel(input, table):
    orig_shape = input.shape
    B = input.size
    idx = input.reshape(NW, B // NW // CHUNK, CHUNK).astype(jnp.int32)
    out = _build(B)(idx, table)
    return out.reshape(orig_shape + (D,))
